# baseline (speedup 1.0000x reference)
import functools

import jax
import jax.numpy as jnp
from jax import lax
from jax.experimental import pallas as pl
from jax.experimental.pallas import tpu as pltpu
from jax.experimental.pallas import tpu_sc as plsc

_NC = 2
_NS = 16
_LANES = 16
_CHUNK = 128
_NBUF = 4
_NMETA = 8


@functools.partial(jax.jit, static_argnames=("npad", "dh", "epw"))
def _sc_spmm(xh, meta, zblk, *, npad, dh, epw):
    nchunks = epw // _CHUNK
    rps = npad // _NS

    mesh = plsc.VectorSubcoreMesh(core_axis_name="c", subcore_axis_name="s")

    @functools.partial(
        pl.kernel,
        mesh=mesh,
        out_type=jax.ShapeDtypeStruct((_NC * npad, dh), jnp.float32),
        scratch_types=[
            pltpu.VMEM((_NMETA, 3, _CHUNK), jnp.int32),
            pltpu.VMEM((_NBUF, _CHUNK, dh), jnp.bfloat16),
            pltpu.VMEM((_NBUF, _CHUNK, dh), jnp.float32),
            pltpu.VMEM_SHARED((npad, dh), jnp.bfloat16),
            pltpu.VMEM_SHARED((npad, dh), jnp.float32),
            [pltpu.SemaphoreType.DMA] * _NMETA,
            [pltpu.SemaphoreType.DMA] * _NBUF,
            [pltpu.SemaphoreType.DMA] * _NBUF,
        ],
        compiler_params=pltpu.CompilerParams(use_tc_tiling_on_sc=False,
                                             needs_layout_passes=False),
    )
    def k(xh_hbm, meta_hbm, z_hbm, out_hbm, mb, rows, rowsf, xs, acc,
          msems, gsems, ssems):
        c = lax.axis_index("c")
        s = lax.axis_index("s")

        pltpu.sync_copy(xh_hbm.at[c, pl.ds(s * rps, rps)],
                        xs.at[pl.ds(s * rps, rps)])
        pltpu.sync_copy(z_hbm, acc.at[pl.ds(s * rps, rps)])
        plsc.subcore_barrier()

        def start_meta(i, m):
            pltpu.async_copy(meta_hbm.at[s, i], mb.at[m], msems[m])

        def wait_meta(m):
            pltpu.make_async_copy(meta_hbm.at[0, 0], mb.at[m],
                                  msems[m]).wait()

        def start_gather(b, m):
            pltpu.async_copy(xs.at[mb.at[m, 0]], rows.at[b], gsems[b])

        def wait_gather(b):
            pltpu.make_async_copy(xs.at[mb.at[0, 0]],
                                  rows.at[b], gsems[b]).wait()

        def start_scatter(b, m):
            pltpu.async_copy(rowsf.at[b], acc.at[mb.at[m, 1]], ssems[b],
                             add=True)

        def wait_scatter(b):
            pltpu.make_async_copy(rowsf.at[b], acc.at[mb.at[0, 1]],
                                  ssems[b]).wait()

        def scale(b, m):
            @plsc.parallel_loop(0, _CHUNK, _LANES, unroll=2)
            def _scale(q):
                wvec = plsc.bitcast(mb[m, 2, pl.ds(q, _LANES)], jnp.float32)
                for e in range(_LANES):
                    wj = lax.gather(
                        wvec, jnp.full((_LANES, 1), e, jnp.int32),
                        lax.GatherDimensionNumbers(
                            offset_dims=(), collapsed_slice_dims=(0,),
                            start_index_map=(0,)),
                        (1,),
                        mode=lax.GatherScatterMode.PROMISE_IN_BOUNDS)
                    for t in range(dh // (2 * _LANES)):
                        v = rows[b, q + e, pl.ds(t * 2 * _LANES, 2 * _LANES)]
                        lo, hi = plsc.unpack(
                            v, format=plsc.PackFormat.INTERLEAVED)
                        base = t * 2 * _LANES
                        rowsf[b, q + e, pl.ds(base, _LANES)] = lo * wj
                        rowsf[b, q + e, pl.ds(base + _LANES, _LANES)] = \
                            hi * wj

        start_meta(0, 0)
        start_meta(1, 1)
        wait_meta(0)
        start_gather(0, 0)

        @pl.loop(0, nchunks, step=_NMETA)
        def _ring(i0):
            for j in range(_NMETA):
                i = i0 + j
                m = j
                b = j % _NBUF
                nm = (j + 2) % _NMETA
                gm = (j + 1) % _NMETA
                nb = (j + 1) % _NBUF

                wait_gather(b)

                @pl.when(i + 2 < nchunks)
                def _():
                    start_meta(i + 2, nm)

                @pl.when(i + 1 < nchunks)
                def _():
                    wait_meta(gm)
                    start_gather(nb, gm)

                @pl.when(i >= _NBUF)
                def _():
                    wait_scatter(b)

                scale(b, m)
                start_scatter(b, m)

        for t in range(_NBUF):
            wait_scatter((nchunks - _NBUF + t) % _NBUF)

        plsc.subcore_barrier()
        pltpu.sync_copy(acc.at[pl.ds(s * rps, rps)],
                        out_hbm.at[pl.ds(c * npad + s * rps, rps)])

    return k(xh, meta, zblk)


def kernel(seq, edge_index, edge_weight):
    x = seq[0]
    n, d = x.shape
    e = edge_weight.shape[0]
    dh = d // 2

    dst = edge_index[0].astype(jnp.int32)
    src = edge_index[1].astype(jnp.int32)
    w = edge_weight.astype(jnp.float32)

    quantum = _NS * _CHUNK * _NMETA
    epad = -(-e // quantum) * quantum
    pad = epad - e
    if pad:
        src = jnp.concatenate([src, jnp.zeros((pad,), jnp.int32)])
        dst = jnp.concatenate([dst, jnp.zeros((pad,), jnp.int32)])
        w = jnp.concatenate([w, jnp.zeros((pad,), jnp.float32)])
    epw = epad // _NS
    nchunks = epw // _CHUNK

    wi = lax.bitcast_convert_type(w, jnp.int32)
    meta = jnp.stack(
        [t.reshape(_NS, nchunks, _CHUNK) for t in (src, dst, wi)], axis=2)

    ci = jnp.arange(d)
    g, r = ci // 32, ci % 32
    perm = g * 32 + jnp.where(r % 2 == 0, r // 2, 16 + r // 2)
    xp = x[:, perm].astype(jnp.bfloat16)

    npad = -(-n // (_NS * 8)) * (_NS * 8)
    xh = jnp.zeros((_NC, npad, dh), jnp.bfloat16)
    xh = xh.at[0, :n].set(xp[:, :dh]).at[1, :n].set(xp[:, dh:])
    zblk = jnp.zeros((npad // _NS, dh), jnp.float32)

    out2 = _sc_spmm(xh, meta, zblk, npad=npad, dh=dh, epw=epw)
    out = jnp.concatenate([out2[:n], out2[npad:npad + n]], axis=1)
    return out[None]


# speedup vs baseline: 1.3869x; 1.3869x over previous
import functools

import jax
import jax.numpy as jnp
from jax import lax
from jax.experimental import pallas as pl
from jax.experimental.pallas import tpu as pltpu
from jax.experimental.pallas import tpu_sc as plsc

_NC = 2
_NS = 16
_LANES = 16
_CHUNK = 128
_NBUF = 4
_NMETA = 8


@functools.partial(jax.jit, static_argnames=("npad", "dh", "epw"))
def _sc_spmm(xh, meta, zblk, *, npad, dh, epw):
    nchunks = epw // _CHUNK
    rps = npad // _NS

    mesh = plsc.VectorSubcoreMesh(core_axis_name="c", subcore_axis_name="s")

    @functools.partial(
        pl.kernel,
        mesh=mesh,
        out_type=jax.ShapeDtypeStruct((_NC * npad, dh), jnp.float32),
        scratch_types=[
            pltpu.VMEM((_NMETA, 3, _CHUNK), jnp.int32),
            pltpu.VMEM((_NBUF, _CHUNK, dh), jnp.float32),
            pltpu.VMEM_SHARED((npad, dh), jnp.float32),
            pltpu.VMEM_SHARED((npad, dh), jnp.float32),
            [pltpu.SemaphoreType.DMA] * _NMETA,
            [pltpu.SemaphoreType.DMA] * _NBUF,
            [pltpu.SemaphoreType.DMA] * _NBUF,
        ],
        compiler_params=pltpu.CompilerParams(use_tc_tiling_on_sc=False,
                                             needs_layout_passes=False),
    )
    def k(xh_hbm, meta_hbm, z_hbm, out_hbm, mb, rows, xs, acc,
          msems, gsems, ssems):
        c = lax.axis_index("c")
        s = lax.axis_index("s")

        pltpu.sync_copy(xh_hbm.at[c, pl.ds(s * rps, rps)],
                        xs.at[pl.ds(s * rps, rps)])
        pltpu.sync_copy(z_hbm, acc.at[pl.ds(s * rps, rps)])
        plsc.subcore_barrier()

        def start_meta(i, m):
            pltpu.async_copy(meta_hbm.at[s, i], mb.at[m], msems[m])

        def wait_meta(m):
            pltpu.make_async_copy(meta_hbm.at[0, 0], mb.at[m],
                                  msems[m]).wait()

        def start_gather(b, m):
            pltpu.async_copy(xs.at[mb.at[m, 0]], rows.at[b], gsems[b])

        def wait_gather(b):
            pltpu.make_async_copy(xs.at[mb.at[0, 0]],
                                  rows.at[b], gsems[b]).wait()

        def start_scatter(b, m):
            pltpu.async_copy(rows.at[b], acc.at[mb.at[m, 1]], ssems[b],
                             add=True)

        def wait_scatter(b):
            pltpu.make_async_copy(rows.at[b], acc.at[mb.at[0, 1]],
                                  ssems[b]).wait()

        def scale(b, m):
            @plsc.parallel_loop(0, _CHUNK, _LANES, unroll=2)
            def _scale(q):
                wvec = plsc.bitcast(mb[m, 2, pl.ds(q, _LANES)], jnp.float32)
                for e in range(_LANES):
                    wj = lax.gather(
                        wvec, jnp.full((_LANES, 1), e, jnp.int32),
                        lax.GatherDimensionNumbers(
                            offset_dims=(), collapsed_slice_dims=(0,),
                            start_index_map=(0,)),
                        (1,),
                        mode=lax.GatherScatterMode.PROMISE_IN_BOUNDS)
                    for kk in range(dh // _LANES):
                        sl = (b, q + e, pl.ds(kk * _LANES, _LANES))
                        rows[sl] = rows[sl] * wj

        start_meta(0, 0)
        start_meta(1, 1)
        start_meta(2, 2)
        wait_meta(0)
        start_gather(0, 0)
        wait_meta(1)
        start_gather(1, 1)

        @pl.loop(0, nchunks, step=_NMETA)
        def _ring(i0):
            for j in range(_NMETA):
                i = i0 + j
                m = j
                b = j % _NBUF
                nm = (j + 3) % _NMETA
                gm = (j + 2) % _NMETA
                gb = (j + 2) % _NBUF

                wait_gather(b)

                @pl.when(i + 3 < nchunks)
                def _():
                    start_meta(i + 3, nm)

                @pl.when(i >= _NBUF - 2)
                def _():
                    wait_scatter(gb)

                @pl.when(i + 2 < nchunks)
                def _():
                    wait_meta(gm)
                    start_gather(gb, gm)

                scale(b, m)
                start_scatter(b, m)

        for t in range(_NBUF - 2):
            wait_scatter((nchunks - (_NBUF - 2) + t) % _NBUF)

        plsc.subcore_barrier()
        pltpu.sync_copy(acc.at[pl.ds(s * rps, rps)],
                        out_hbm.at[pl.ds(c * npad + s * rps, rps)])

    return k(xh, meta, zblk)


def kernel(seq, edge_index, edge_weight):
    x = seq[0]
    n, d = x.shape
    e = edge_weight.shape[0]
    dh = d // 2

    dst = edge_index[0].astype(jnp.int32)
    src = edge_index[1].astype(jnp.int32)
    w = edge_weight.astype(jnp.float32)

    quantum = _NS * _CHUNK * _NMETA
    epad = -(-e // quantum) * quantum
    pad = epad - e
    if pad:
        src = jnp.concatenate([src, jnp.zeros((pad,), jnp.int32)])
        dst = jnp.concatenate([dst, jnp.zeros((pad,), jnp.int32)])
        w = jnp.concatenate([w, jnp.zeros((pad,), jnp.float32)])
    epw = epad // _NS
    nchunks = epw // _CHUNK

    wi = lax.bitcast_convert_type(w, jnp.int32)
    meta = jnp.stack(
        [t.reshape(_NS, nchunks, _CHUNK) for t in (src, dst, wi)], axis=2)

    npad = -(-n // (_NS * 8)) * (_NS * 8)
    xh = jnp.zeros((_NC, npad, dh), jnp.float32)
    xh = xh.at[0, :n].set(x[:, :dh]).at[1, :n].set(x[:, dh:])
    zblk = jnp.zeros((npad // _NS, dh), jnp.float32)

    out2 = _sc_spmm(xh, meta, zblk, npad=npad, dh=dh, epw=epw)
    out = jnp.concatenate([out2[:n], out2[npad:npad + n]], axis=1)
    return out[None]


# speedup vs baseline: 1.3921x; 1.0038x over previous
import functools

import jax
import jax.numpy as jnp
from jax import lax
from jax.experimental import pallas as pl
from jax.experimental.pallas import tpu as pltpu
from jax.experimental.pallas import tpu_sc as plsc

_NC = 2
_NS = 16
_LANES = 16
_CHUNK = 128
_NBUF = 4
_NMETA = 8


@functools.partial(jax.jit, static_argnames=("npad", "dh", "epw"))
def _sc_spmm(xh, meta, zblk, *, npad, dh, epw):
    nchunks = epw // _CHUNK
    rps = npad // _NS

    mesh = plsc.VectorSubcoreMesh(core_axis_name="c", subcore_axis_name="s")

    @functools.partial(
        pl.kernel,
        mesh=mesh,
        out_type=jax.ShapeDtypeStruct((_NC * npad, dh), jnp.float32),
        scratch_types=[
            pltpu.VMEM((_NMETA, 3, _CHUNK), jnp.int32),
            pltpu.VMEM((_NBUF, _CHUNK, dh), jnp.float32),
            pltpu.VMEM_SHARED((npad, dh), jnp.float32),
            pltpu.VMEM_SHARED((npad, dh), jnp.float32),
            [pltpu.SemaphoreType.DMA] * _NMETA,
            [pltpu.SemaphoreType.DMA] * _NBUF,
            [pltpu.SemaphoreType.DMA] * _NBUF,
        ],
        compiler_params=pltpu.CompilerParams(use_tc_tiling_on_sc=False,
                                             needs_layout_passes=False),
    )
    def k(xh_hbm, meta_hbm, z_hbm, out_hbm, mb, rows, xs, acc,
          msems, gsems, ssems):
        c = lax.axis_index("c")
        s = lax.axis_index("s")

        cp_x = pltpu.async_copy(xh_hbm.at[c, pl.ds(s * rps, rps)],
                                xs.at[pl.ds(s * rps, rps)], gsems[0])
        cp_z = pltpu.async_copy(z_hbm, acc.at[pl.ds(s * rps, rps)], gsems[1])
        cp_x.wait()
        cp_z.wait()
        plsc.subcore_barrier()

        def start_meta(i, m):
            pltpu.async_copy(meta_hbm.at[s, i], mb.at[m], msems[m])

        def wait_meta(m):
            pltpu.make_async_copy(meta_hbm.at[0, 0], mb.at[m],
                                  msems[m]).wait()

        def start_gather(b, m):
            pltpu.async_copy(xs.at[mb.at[m, 0]], rows.at[b], gsems[b])

        def wait_gather(b):
            pltpu.make_async_copy(xs.at[mb.at[0, 0]],
                                  rows.at[b], gsems[b]).wait()

        def start_scatter(b, m):
            pltpu.async_copy(rows.at[b], acc.at[mb.at[m, 1]], ssems[b],
                             add=True)

        def wait_scatter(b):
            pltpu.make_async_copy(rows.at[b], acc.at[mb.at[0, 1]],
                                  ssems[b]).wait()

        def scale(b, m):
            @plsc.parallel_loop(0, _CHUNK, _LANES, unroll=4)
            def _scale(q):
                wvec = plsc.bitcast(mb[m, 2, pl.ds(q, _LANES)], jnp.float32)
                for e in range(_LANES):
                    wj = lax.gather(
                        wvec, jnp.full((_LANES, 1), e, jnp.int32),
                        lax.GatherDimensionNumbers(
                            offset_dims=(), collapsed_slice_dims=(0,),
                            start_index_map=(0,)),
                        (1,),
                        mode=lax.GatherScatterMode.PROMISE_IN_BOUNDS)
                    for kk in range(dh // _LANES):
                        sl = (b, q + e, pl.ds(kk * _LANES, _LANES))
                        rows[sl] = rows[sl] * wj

        start_meta(0, 0)
        start_meta(1, 1)
        start_meta(2, 2)
        wait_meta(0)
        start_gather(0, 0)
        wait_meta(1)
        start_gather(1, 1)

        @pl.loop(0, nchunks, step=_NMETA)
        def _ring(i0):
            for j in range(_NMETA):
                i = i0 + j
                m = j
                b = j % _NBUF
                nm = (j + 3) % _NMETA
                gm = (j + 2) % _NMETA
                gb = (j + 2) % _NBUF

                wait_gather(b)

                @pl.when(i + 3 < nchunks)
                def _():
                    start_meta(i + 3, nm)

                @pl.when(i >= _NBUF - 2)
                def _():
                    wait_scatter(gb)

                @pl.when(i + 2 < nchunks)
                def _():
                    wait_meta(gm)
                    start_gather(gb, gm)

                scale(b, m)
                start_scatter(b, m)

        for t in range(_NBUF - 2):
            wait_scatter((nchunks - (_NBUF - 2) + t) % _NBUF)

        plsc.subcore_barrier()
        pltpu.sync_copy(acc.at[pl.ds(s * rps, rps)],
                        out_hbm.at[pl.ds(c * npad + s * rps, rps)])

    return k(xh, meta, zblk)


def kernel(seq, edge_index, edge_weight):
    x = seq[0]
    n, d = x.shape
    e = edge_weight.shape[0]
    dh = d // 2

    dst = edge_index[0].astype(jnp.int32)
    src = edge_index[1].astype(jnp.int32)
    w = edge_weight.astype(jnp.float32)

    quantum = _NS * _CHUNK * _NMETA
    epad = -(-e // quantum) * quantum
    pad = epad - e
    if pad:
        src = jnp.concatenate([src, jnp.zeros((pad,), jnp.int32)])
        dst = jnp.concatenate([dst, jnp.zeros((pad,), jnp.int32)])
        w = jnp.concatenate([w, jnp.zeros((pad,), jnp.float32)])
    epw = epad // _NS
    nchunks = epw // _CHUNK

    wi = lax.bitcast_convert_type(w, jnp.int32)
    meta = jnp.stack(
        [t.reshape(_NS, nchunks, _CHUNK) for t in (src, dst, wi)], axis=2)

    npad = -(-n // (_NS * 8)) * (_NS * 8)
    xh = jnp.zeros((_NC, npad, dh), jnp.float32)
    xh = xh.at[0, :n].set(x[:, :dh]).at[1, :n].set(x[:, dh:])
    zblk = jnp.zeros((npad // _NS, dh), jnp.float32)

    out2 = _sc_spmm(xh, meta, zblk, npad=npad, dh=dh, epw=epw)
    out = jnp.concatenate([out2[:n], out2[npad:npad + n]], axis=1)
    return out[None]
